# baseline (speedup 1.0000x reference)
import functools

import jax
import jax.numpy as jnp
from jax import lax
from jax.experimental import pallas as pl
from jax.experimental.pallas import tpu as pltpu
from jax.experimental.pallas import tpu_sc as plsc

BATCH = 16384
EMB = 64
PAD = 2 * EMB
HID = 10
NUSER = 1000000
NMOVIE = 100000

_info = plsc.get_sparse_core_info()
_NC, _NS = _info.num_cores, _info.num_subcores
NW = _NC * _NS

BLK_R = 1024
NBLK_U = 976
BOUND_U = NBLK_U * BLK_R
NBLK_M = 97
BOUND_M = NBLK_M * BLK_R
FLUSH = 64
STG = BATCH + NW
SENT = 0x7FFFFFF0
TRASH = BATCH + 16
IDXC = 4096
HBLK = BLK_R // 2


def _sweep_table(tT_hbm, idx_hbm, o_hbm, b0, nb_full, lo, hi, load_max,
                 allidx_v, cand_r, cand_p, hit_h, blkA, blkB, flush_v,
                 flush_p, sem, semA, semB, wid):
    iota = lax.iota(jnp.int32, 16)

    n_c = jnp.int32(0)
    for sc in range(BATCH // IDXC):
        pltpu.sync_copy(idx_hbm.at[pl.ds(sc * IDXC, IDXC)], allidx_v)

        def cscan(ch, n_c, sc=sc):
            chunk = allidx_v[pl.ds(ch * 16, 16)]
            m = (chunk >= lo) & (chunk < hi)
            pref = plsc.cumsum(m.astype(jnp.int32))
            dst = jnp.where(m, n_c + pref - 1, TRASH)
            plsc.store_scatter(cand_r, [dst], chunk)
            plsc.store_scatter(cand_p, [dst], (sc * IDXC + ch * 16) + iota)
            return n_c + pref[15]

        n_c = lax.fori_loop(0, IDXC // 16, cscan, n_c)
    cand_r[pl.ds(n_c, 16)] = jnp.full((16,), SENT, jnp.int32)
    cand_p[pl.ds(n_c, 16)] = jnp.full((16,), -1, jnp.int32)
    n_cg = (n_c + 15) // 16

    def hscan(r0):
        def f(g, n_h):
            cr = cand_r[pl.ds(g * 16, 16)]
            cp = cand_p[pl.ds(g * 16, 16)]
            m = (cr >= r0) & (cr < r0 + HBLK)
            pref = plsc.cumsum(m.astype(jnp.int32))
            dst = jnp.where(m, n_h + pref - 1, TRASH)
            plsc.store_scatter(hit_h, [dst], cp * BLK_R + (cr - r0))
            return n_h + pref[15]

        return lax.fori_loop(0, n_cg, f, jnp.int32(0))

    def proc(n_h, buf):
        hit_h[pl.ds(n_h, 16)] = jnp.full((16,), -1, jnp.int32)

        def hproc(g, carry):
            h16 = hit_h[pl.ds(g * 16, 16)]
            hr = h16 & (BLK_R - 1)
            hp = h16 // BLK_R
            flush_p[0, pl.ds(0, 16)] = jnp.where(h16 >= 0, hp, BATCH + wid)
            for l in range(16):
                rloc = hr[l]
                col = jnp.full((16,), rloc, jnp.int32)
                for k in range(4):
                    vals = plsc.load_gather(buf, [iota + 16 * k, col])
                    flush_v[l, pl.ds(16 * k, 16)] = vals
            pltpu.async_copy(flush_v, o_hbm.at[flush_p.at[0]], sem).wait()
            return carry

        lax.fori_loop(0, (n_h + 15) // 16, hproc, jnp.int32(0))

    def fire(r0, buf, lsem):
        r0c = jnp.minimum(r0, load_max)
        pltpu.async_copy(tT_hbm.at[:, pl.ds(r0c, HBLK)], buf, lsem)

    def drain(buf, lsem):
        pltpu.make_async_copy(tT_hbm.at[:, pl.ds(0, HBLK)], buf, lsem).wait()

    fire(b0 * BLK_R, blkA, semA)

    def body(bb, carry):
        r0 = (b0 + bb) * BLK_R
        fire(r0 + HBLK, blkB, semB)
        drain(blkA, semA)
        proc(hscan(r0), blkA)
        fire(r0 + BLK_R, blkA, semA)
        drain(blkB, semB)
        proc(hscan(r0 + HBLK), blkB)
        return carry

    lax.fori_loop(0, nb_full, body, jnp.int32(0))
    drain(blkA, semA)

def _sc_gather_body(user_hbm, movie_hbm, utT_hbm, mtT_hbm, ue_hbm, me_hbm,
                    allidx_v, cand_r, cand_p, hit_h, blkA, blkB, flush_v,
                    flush_p, sem, semA, semB):
    wid = lax.axis_index("s") * _NC + lax.axis_index("c")

    for s in range(16):
        for k in range(4):
            flush_v[s, pl.ds(EMB + 16 * k, 16)] = jnp.zeros((16,), jnp.float32)

    b0u = jnp.where(wid < 16, wid * 31, wid * 30 + 16)
    nbu = jnp.where(wid < 16, 31, 30)
    _sweep_table(utT_hbm, user_hbm, ue_hbm, b0u, nbu,
                 b0u * BLK_R, (b0u + nbu) * BLK_R, BOUND_U - HBLK,
                 allidx_v, cand_r, cand_p, hit_h, blkA, blkB, flush_v,
                 flush_p, sem, semA, semB, wid)

    b0m = wid * 3 + jnp.minimum(wid, 1)
    nbm = jnp.where(wid == 0, 4, 3)
    _sweep_table(mtT_hbm, movie_hbm, me_hbm, b0m, nbm,
                 b0m * BLK_R, (b0m + nbm) * BLK_R, BOUND_M - HBLK,
                 allidx_v, cand_r, cand_p, hit_h, blkA, blkB, flush_v,
                 flush_p, sem, semA, semB, wid)


_sc_gather = functools.partial(
    pl.kernel,
    out_type=[
        jax.ShapeDtypeStruct((STG, PAD), jnp.float32),
        jax.ShapeDtypeStruct((STG, PAD), jnp.float32),
    ],
    mesh=plsc.VectorSubcoreMesh(core_axis_name="c", subcore_axis_name="s"),
    scratch_types=[
        pltpu.VMEM((IDXC,), jnp.int32),
        pltpu.VMEM((BATCH + 32,), jnp.int32),
        pltpu.VMEM((BATCH + 32,), jnp.int32),
        pltpu.VMEM((BATCH + 32,), jnp.int32),
        pltpu.VMEM((EMB, HBLK), jnp.float32),
        pltpu.VMEM((EMB, HBLK), jnp.float32),
        pltpu.VMEM((16, PAD), jnp.float32),
        pltpu.VMEM((1, 16), jnp.int32),
        pltpu.SemaphoreType.DMA,
        pltpu.SemaphoreType.DMA,
        pltpu.SemaphoreType.DMA,
    ],
    compiler_params=pltpu.CompilerParams(needs_layout_passes=False),
)(_sc_gather_body)


def _mlp_body(ue_ref, me_ref, w1u_ref, w1m_ref, b1_ref, w2_ref, b2_ref,
              out_ref):
    ue = jnp.maximum(ue_ref[...], 0.0)
    me = jnp.maximum(me_ref[...], 0.0)
    h = (
        jnp.dot(ue, w1u_ref[...], preferred_element_type=jnp.float32)
        + jnp.dot(me, w1m_ref[...], preferred_element_type=jnp.float32)
        + b1_ref[...]
    )
    h = jnp.maximum(h, 0.0)
    out_ref[...] = (
        jnp.dot(h, w2_ref[...], preferred_element_type=jnp.float32)
        + b2_ref[...]
    )


def _mlp(ue, me, w1u, w1m, b1, w2, b2):
    blk = 2048
    grid = (BATCH // blk,)
    return pl.pallas_call(
        _mlp_body,
        grid=grid,
        in_specs=[
            pl.BlockSpec((blk, PAD), lambda i: (i, 0)),
            pl.BlockSpec((blk, PAD), lambda i: (i, 0)),
            pl.BlockSpec((PAD, HID), lambda i: (0, 0)),
            pl.BlockSpec((PAD, HID), lambda i: (0, 0)),
            pl.BlockSpec((1, HID), lambda i: (0, 0)),
            pl.BlockSpec((HID, 1), lambda i: (0, 0)),
            pl.BlockSpec((1, 1), lambda i: (0, 0)),
        ],
        out_specs=pl.BlockSpec((blk, 1), lambda i: (i, 0)),
        out_shape=jax.ShapeDtypeStruct((BATCH, 1), jnp.float32),
    )(ue, me, w1u, w1m, b1, w2, b2)


def kernel(user, movie, user_table, movie_table, W1, b1, W2, b2):
    user = user.astype(jnp.int32)
    movie = movie.astype(jnp.int32)
    utT = user_table.T
    mtT = movie_table.T
    ue_st, me_st = _sc_gather(user, movie, utT, mtT)
    ue_st = ue_st[:BATCH]
    me_st = me_st[:BATCH]

    tail_u = user_table[BOUND_U:]
    tail_m = movie_table[BOUND_M:]
    mu = user >= BOUND_U
    mm = movie >= BOUND_M
    pu = jnp.pad(jnp.take(tail_u, jnp.clip(user - BOUND_U, 0, NUSER - BOUND_U - 1), axis=0),
                 ((0, 0), (0, PAD - EMB)))
    pm = jnp.pad(jnp.take(tail_m, jnp.clip(movie - BOUND_M, 0, NMOVIE - BOUND_M - 1), axis=0),
                 ((0, 0), (0, PAD - EMB)))
    ue_st = jnp.where(mu[:, None], pu, ue_st)
    me_st = jnp.where(mm[:, None], pm, me_st)

    z = jnp.zeros((EMB, HID), jnp.float32)
    w1u = jnp.concatenate([W1[:, :EMB].T, z], axis=0)
    w1m = jnp.concatenate([W1[:, EMB:].T, z], axis=0)
    return _mlp(ue_st, me_st, w1u, w1m,
                b1.reshape(1, HID), W2.T, b2.reshape(1, 1))


# speedup vs baseline: 1.0904x; 1.0904x over previous
import functools

import jax
import jax.numpy as jnp
from jax import lax
from jax.experimental import pallas as pl
from jax.experimental.pallas import tpu as pltpu
from jax.experimental.pallas import tpu_sc as plsc

BATCH = 16384
EMB = 64
PAD = 2 * EMB
HID = 10
NUSER = 1000000
NMOVIE = 100000

_info = plsc.get_sparse_core_info()
_NC, _NS = _info.num_cores, _info.num_subcores
NW = _NC * _NS

BLK_R = 1024
NBLK_U = 976
BOUND_U = NBLK_U * BLK_R
NBLK_M = 97
BOUND_M = NBLK_M * BLK_R
SWEPT_U = 999936
SWEPT_M = 99968
FLUSH = 64
STG = BATCH + NW
SENT = 0x7FFFFFF0
TRASH = BATCH + 16
IDXC = 4096
HBLK = BLK_R // 2


def _sweep_table(tT_hbm, idx_hbm, o_hbm, b0, nb_full, lo, hi, load_max,
                 tail_blocks,
                 allidx_v, cand_r, cand_p, hit_h, blkA, blkB, flush_v,
                 flush_p, sem, semA, semB, wid):
    iota = lax.iota(jnp.int32, 16)

    n_c = jnp.int32(0)
    for sc in range(BATCH // IDXC):
        pltpu.sync_copy(idx_hbm.at[pl.ds(sc * IDXC, IDXC)], allidx_v)

        def cscan(ch, n_c, sc=sc):
            chunk = allidx_v[pl.ds(ch * 16, 16)]
            m = (chunk >= lo) & (chunk < hi)
            pref = plsc.cumsum(m.astype(jnp.int32))
            dst = jnp.where(m, n_c + pref - 1, TRASH)
            plsc.store_scatter(cand_r, [dst], chunk)
            plsc.store_scatter(cand_p, [dst], (sc * IDXC + ch * 16) + iota)
            return n_c + pref[15]

        n_c = lax.fori_loop(0, IDXC // 16, cscan, n_c)
    cand_r[pl.ds(n_c, 16)] = jnp.full((16,), SENT, jnp.int32)
    cand_p[pl.ds(n_c, 16)] = jnp.full((16,), -1, jnp.int32)
    n_cg = (n_c + 15) // 16

    def hscan(r0, width=HBLK):
        def f(g, n_h):
            cr = cand_r[pl.ds(g * 16, 16)]
            cp = cand_p[pl.ds(g * 16, 16)]
            m = (cr >= r0) & (cr < r0 + width)
            pref = plsc.cumsum(m.astype(jnp.int32))
            dst = jnp.where(m, n_h + pref - 1, TRASH)
            plsc.store_scatter(hit_h, [dst], cp * BLK_R + (cr - r0))
            return n_h + pref[15]

        return lax.fori_loop(0, n_cg, f, jnp.int32(0))

    def proc(n_h, buf):
        hit_h[pl.ds(n_h, 16)] = jnp.full((16,), -1, jnp.int32)

        def hproc(g, carry):
            h16 = hit_h[pl.ds(g * 16, 16)]
            hr = jnp.where(h16 >= 0, h16 & (BLK_R - 1), 0)
            hp = h16 // BLK_R
            flush_p[0, pl.ds(0, 16)] = jnp.where(h16 >= 0, hp, BATCH + wid)
            for l in range(16):
                rloc = hr[l]
                col = jnp.full((16,), rloc, jnp.int32)
                for k in range(4):
                    vals = plsc.load_gather(buf, [iota + 16 * k, col])
                    flush_v[l, pl.ds(16 * k, 16)] = vals
            pltpu.async_copy(flush_v, o_hbm.at[flush_p.at[0]], sem).wait()
            return carry

        lax.fori_loop(0, (n_h + 15) // 16, hproc, jnp.int32(0))

    def fire(r0, buf, lsem):
        r0c = jnp.minimum(r0, load_max)
        pltpu.async_copy(tT_hbm.at[:, pl.ds(r0c, HBLK)], buf, lsem)

    def drain(buf, lsem):
        pltpu.make_async_copy(tT_hbm.at[:, pl.ds(0, HBLK)], buf, lsem).wait()

    fire(b0 * BLK_R, blkA, semA)

    def body(bb, carry):
        r0 = (b0 + bb) * BLK_R
        fire(r0 + HBLK, blkB, semB)
        drain(blkA, semA)
        proc(hscan(r0), blkA)
        fire(r0 + BLK_R, blkA, semA)
        drain(blkB, semB)
        proc(hscan(r0 + HBLK), blkB)
        return carry

    lax.fori_loop(0, nb_full, body, jnp.int32(0))
    drain(blkA, semA)

    if tail_blocks:

        @pl.when(wid == NW - 1)
        def _():
            for r0t, wt in tail_blocks:
                pltpu.sync_copy(tT_hbm.at[:, pl.ds(r0t, wt)],
                                blkA.at[:, pl.ds(0, wt)])
                proc(hscan(jnp.int32(r0t), wt), blkA)

def _sc_gather_body(user_hbm, movie_hbm, utT_hbm, mtT_hbm, ue_hbm, me_hbm,
                    allidx_v, cand_r, cand_p, hit_h, blkA, blkB, flush_v,
                    flush_p, sem, semA, semB):
    wid = lax.axis_index("s") * _NC + lax.axis_index("c")

    for s in range(16):
        for k in range(4):
            flush_v[s, pl.ds(EMB + 16 * k, 16)] = jnp.zeros((16,), jnp.float32)

    b0u = jnp.where(wid < 16, wid * 31, wid * 30 + 16)
    nbu = jnp.where(wid < 16, 31, 30)
    hiu = jnp.where(wid == NW - 1, SWEPT_U, (b0u + nbu) * BLK_R)
    _sweep_table(utT_hbm, user_hbm, ue_hbm, b0u, nbu,
                 b0u * BLK_R, hiu, BOUND_U - HBLK,
                 ((BOUND_U, 512),),
                 allidx_v, cand_r, cand_p, hit_h, blkA, blkB, flush_v,
                 flush_p, sem, semA, semB, wid)

    b0m = wid * 3 + jnp.minimum(wid, 1)
    nbm = jnp.where(wid == 0, 4, 3)
    him = jnp.where(wid == NW - 1, SWEPT_M, (b0m + nbm) * BLK_R)
    _sweep_table(mtT_hbm, movie_hbm, me_hbm, b0m, nbm,
                 b0m * BLK_R, him, BOUND_M - HBLK,
                 ((BOUND_M, 512), (BOUND_M + 512, 128)),
                 allidx_v, cand_r, cand_p, hit_h, blkA, blkB, flush_v,
                 flush_p, sem, semA, semB, wid)


_sc_gather = functools.partial(
    pl.kernel,
    out_type=[
        jax.ShapeDtypeStruct((STG, PAD), jnp.float32),
        jax.ShapeDtypeStruct((STG, PAD), jnp.float32),
    ],
    mesh=plsc.VectorSubcoreMesh(core_axis_name="c", subcore_axis_name="s"),
    scratch_types=[
        pltpu.VMEM((IDXC,), jnp.int32),
        pltpu.VMEM((BATCH + 32,), jnp.int32),
        pltpu.VMEM((BATCH + 32,), jnp.int32),
        pltpu.VMEM((BATCH + 32,), jnp.int32),
        pltpu.VMEM((EMB, HBLK), jnp.float32),
        pltpu.VMEM((EMB, HBLK), jnp.float32),
        pltpu.VMEM((16, PAD), jnp.float32),
        pltpu.VMEM((1, 16), jnp.int32),
        pltpu.SemaphoreType.DMA,
        pltpu.SemaphoreType.DMA,
        pltpu.SemaphoreType.DMA,
    ],
    compiler_params=pltpu.CompilerParams(needs_layout_passes=False),
)(_sc_gather_body)


def _mlp_body(ue_ref, me_ref, w1u_ref, w1m_ref, b1_ref, w2_ref, b2_ref,
              out_ref):
    ue = jnp.maximum(ue_ref[...], 0.0)
    me = jnp.maximum(me_ref[...], 0.0)
    h = (
        jnp.dot(ue, w1u_ref[...], preferred_element_type=jnp.float32)
        + jnp.dot(me, w1m_ref[...], preferred_element_type=jnp.float32)
        + b1_ref[...]
    )
    h = jnp.maximum(h, 0.0)
    out_ref[...] = (
        jnp.dot(h, w2_ref[...], preferred_element_type=jnp.float32)
        + b2_ref[...]
    )


def _mlp(ue, me, w1u, w1m, b1, w2, b2):
    blk = 2048
    grid = (BATCH // blk,)
    return pl.pallas_call(
        _mlp_body,
        grid=grid,
        in_specs=[
            pl.BlockSpec((blk, PAD), lambda i: (i, 0)),
            pl.BlockSpec((blk, PAD), lambda i: (i, 0)),
            pl.BlockSpec((PAD, HID), lambda i: (0, 0)),
            pl.BlockSpec((PAD, HID), lambda i: (0, 0)),
            pl.BlockSpec((1, HID), lambda i: (0, 0)),
            pl.BlockSpec((HID, 1), lambda i: (0, 0)),
            pl.BlockSpec((1, 1), lambda i: (0, 0)),
        ],
        out_specs=pl.BlockSpec((blk, 1), lambda i: (i, 0)),
        out_shape=jax.ShapeDtypeStruct((BATCH, 1), jnp.float32),
    )(ue, me, w1u, w1m, b1, w2, b2)


def kernel(user, movie, user_table, movie_table, W1, b1, W2, b2):
    user = user.astype(jnp.int32)
    movie = movie.astype(jnp.int32)
    utT = user_table.T
    mtT = movie_table.T
    ue_st, me_st = _sc_gather(user, movie, utT, mtT)
    ue_st = ue_st[:BATCH]
    me_st = me_st[:BATCH]

    du = user - SWEPT_U
    dm = movie - SWEPT_M
    oh_u = (du[:, None] == jnp.arange(NUSER - SWEPT_U)[None, :]).astype(jnp.float32)
    oh_m = (dm[:, None] == jnp.arange(NMOVIE - SWEPT_M)[None, :]).astype(jnp.float32)
    pu = jnp.pad(oh_u @ user_table[SWEPT_U:], ((0, 0), (0, PAD - EMB)))
    pm = jnp.pad(oh_m @ movie_table[SWEPT_M:], ((0, 0), (0, PAD - EMB)))
    ue_st = jnp.where((du >= 0)[:, None], pu, ue_st)
    me_st = jnp.where((dm >= 0)[:, None], pm, me_st)

    z = jnp.zeros((EMB, HID), jnp.float32)
    w1u = jnp.concatenate([W1[:, :EMB].T, z], axis=0)
    w1m = jnp.concatenate([W1[:, EMB:].T, z], axis=0)
    return _mlp(ue_st, me_st, w1u, w1m,
                b1.reshape(1, HID), W2.T, b2.reshape(1, 1))
